# initial kernel scaffold (unmeasured)
import jax
import jax.numpy as jnp
from jax import lax
from jax.experimental import pallas as pl
from jax.experimental.pallas import tpu as pltpu

N_DEV = 8


def kernel(A, B):
    m, k_per = A.shape
    _, n = B.shape
    m_per = m // N_DEV

    A16 = A.astype(jnp.bfloat16)
    B16 = B.astype(jnp.bfloat16)

    def body(a_ref, b_ref, out_ref, comm_ref, p_ref, send_sems, recv_sems,
             ready_sems):
        d = lax.axis_index("i")
        left = lax.rem(d + N_DEV - 1, N_DEV)
        right = lax.rem(d + 1, N_DEV)

        barrier_sem = pltpu.get_barrier_semaphore()
        for nbr in (left, right):
            pl.semaphore_signal(
                barrier_sem, inc=1,
                device_id=(nbr,), device_id_type=pl.DeviceIdType.MESH,
            )
        pl.semaphore_wait(barrier_sem, 2)

        def partial_chunk(c):
            a = a_ref[pl.ds(c * m_per, m_per), :]
            return jnp.dot(a, b_ref[:, :], preferred_element_type=jnp.float32)

        comm_ref[0, :, :] = partial_chunk(
            lax.rem(d + N_DEV - 1, N_DEV)).astype(jnp.bfloat16)

        for h in range(N_DEV - 1):
            send_slot = h % 2
            recv_slot = (h + 1) % 2
            if h >= 1:
                pl.semaphore_wait(ready_sems.at[recv_slot], 1)
            rdma = pltpu.make_async_remote_copy(
                src_ref=comm_ref.at[send_slot],
                dst_ref=comm_ref.at[recv_slot],
                send_sem=send_sems.at[send_slot],
                recv_sem=recv_sems.at[recv_slot],
                device_id=(right,),
                device_id_type=pl.DeviceIdType.MESH,
            )
            rdma.start()
            p_ref[:, :] = partial_chunk(lax.rem(d + 2 * N_DEV - 2 - h, N_DEV))
            rdma.wait_send()
            if h <= N_DEV - 3:
                pl.semaphore_signal(
                    ready_sems.at[send_slot], inc=1,
                    device_id=(left,), device_id_type=pl.DeviceIdType.MESH,
                )
            rdma.wait_recv()
            if h < N_DEV - 2:
                comm_ref[recv_slot, :, :] = (
                    comm_ref[recv_slot, :, :].astype(jnp.float32)
                    + p_ref[:, :]
                ).astype(jnp.bfloat16)
            else:
                out_ref[:, :] = (
                    comm_ref[recv_slot, :, :].astype(jnp.float32)
                    + p_ref[:, :]
                )

    return pl.pallas_call(
        body,
        out_shape=jax.ShapeDtypeStruct((m_per, n), jnp.float32),
        in_specs=[
            pl.BlockSpec(memory_space=pltpu.VMEM),
            pl.BlockSpec(memory_space=pltpu.VMEM),
        ],
        out_specs=pl.BlockSpec(memory_space=pltpu.VMEM),
        scratch_shapes=[
            pltpu.VMEM((2, m_per, n), jnp.bfloat16),
            pltpu.VMEM((m_per, n), jnp.float32),
            pltpu.SemaphoreType.DMA((2,)),
            pltpu.SemaphoreType.DMA((2,)),
            pltpu.SemaphoreType.REGULAR((2,)),
        ],
        compiler_params=pltpu.CompilerParams(collective_id=0),
    )(A16, B16)


# baseline (device time: 397614 ns/iter reference)
import jax
import jax.numpy as jnp
from jax import lax
from jax.experimental import pallas as pl
from jax.experimental.pallas import tpu as pltpu

N_DEV = 8


def kernel(A, B):
    m, k_per = A.shape
    _, n = B.shape
    m_per = m // N_DEV

    A16 = A.astype(jnp.bfloat16)
    B16 = B.astype(jnp.bfloat16)

    def body(a_ref, b_ref, out_ref, comm_ref, p_ref, send_sems, recv_sems,
             ready_sems):
        d = lax.axis_index("i")
        left = lax.rem(d + N_DEV - 1, N_DEV)
        right = lax.rem(d + 1, N_DEV)

        barrier_sem = pltpu.get_barrier_semaphore()
        for nbr in (left, right):
            pl.semaphore_signal(
                barrier_sem, inc=1,
                device_id=(nbr,), device_id_type=pl.DeviceIdType.MESH,
            )
        pl.semaphore_wait(barrier_sem, 2)

        def partial_chunk(c):
            a = a_ref[pl.ds(c * m_per, m_per), :]
            return jnp.dot(a, b_ref[:, :], preferred_element_type=jnp.float32)

        comm_ref[0, :, :] = partial_chunk(
            lax.rem(d + N_DEV - 1, N_DEV)).astype(jnp.bfloat16)

        for h in range(N_DEV - 1):
            send_slot = h % 2
            recv_slot = (h + 1) % 2
            if h >= 1:
                pl.semaphore_wait(ready_sems.at[recv_slot], 1)
            rdma = pltpu.make_async_remote_copy(
                src_ref=comm_ref.at[send_slot],
                dst_ref=comm_ref.at[recv_slot],
                send_sem=send_sems.at[send_slot],
                recv_sem=recv_sems.at[recv_slot],
                device_id=(right,),
                device_id_type=pl.DeviceIdType.MESH,
            )
            rdma.start()
            p_ref[:, :] = partial_chunk(lax.rem(d + 2 * N_DEV - 2 - h, N_DEV))
            rdma.wait_send()
            if h <= N_DEV - 3:
                pl.semaphore_signal(
                    ready_sems.at[send_slot], inc=1,
                    device_id=(left,), device_id_type=pl.DeviceIdType.MESH,
                )
            rdma.wait_recv()
            if h < N_DEV - 2:
                comm_ref[recv_slot, :, :] = (
                    comm_ref[recv_slot, :, :].astype(jnp.float32)
                    + p_ref[:, :]
                ).astype(jnp.bfloat16)
            else:
                out_ref[:, :] = (
                    comm_ref[recv_slot, :, :].astype(jnp.float32)
                    + p_ref[:, :]
                )

    return pl.pallas_call(
        body,
        out_shape=jax.ShapeDtypeStruct((m_per, n), jnp.float32),
        in_specs=[
            pl.BlockSpec(memory_space=pltpu.VMEM),
            pl.BlockSpec(memory_space=pltpu.VMEM),
        ],
        out_specs=pl.BlockSpec(memory_space=pltpu.VMEM),
        scratch_shapes=[
            pltpu.VMEM((2, m_per, n), jnp.bfloat16),
            pltpu.VMEM((m_per, n), jnp.float32),
            pltpu.SemaphoreType.DMA((2,)),
            pltpu.SemaphoreType.DMA((2,)),
            pltpu.SemaphoreType.REGULAR((2,)),
        ],
        compiler_params=pltpu.CompilerParams(
            collective_id=0,
            vmem_limit_bytes=64 * 1024 * 1024,
        ),
    )(A16, B16)


# device time: 240331 ns/iter; 1.6544x vs baseline; 1.6544x over previous
import jax
import jax.numpy as jnp
from jax import lax
from jax.experimental import pallas as pl
from jax.experimental.pallas import tpu as pltpu

N_DEV = 8


def kernel(A, B):
    m, k_per = A.shape
    _, n = B.shape
    m_per = m // N_DEV
    nh = n // 2

    A16 = A.astype(jnp.bfloat16)
    B16 = B.astype(jnp.bfloat16)

    def body(a_ref, b_ref, out_ref,
             commA, commB, pA, pB,
             sendA, recvA, readyA, sendB, recvB, readyB):
        d = lax.axis_index("i")

        def ring_map(q):
            return jnp.where(q < 4, q, 11 - q)

        p = ring_map(d)
        right = ring_map(lax.rem(p + 1, N_DEV))
        left = ring_map(lax.rem(p + N_DEV - 1, N_DEV))

        barrier_sem = pltpu.get_barrier_semaphore()
        for nbr in (left, right):
            pl.semaphore_signal(
                barrier_sem, inc=1,
                device_id=(nbr,), device_id_type=pl.DeviceIdType.MESH,
            )
        pl.semaphore_wait(barrier_sem, 2)

        def partial_half(q, col0):
            row = ring_map(q) * m_per
            a = a_ref[pl.ds(row, m_per), :]
            return jnp.dot(a, b_ref[:, col0:col0 + nh],
                           preferred_element_type=jnp.float32)

        commA[0, :, :] = partial_half(
            lax.rem(p + N_DEV - 1, N_DEV), 0).astype(jnp.bfloat16)
        commB[0, :, :] = partial_half(
            lax.rem(p + 1, N_DEV), nh).astype(jnp.bfloat16)

        for h in range(N_DEV - 1):
            s = h % 2
            r = (h + 1) % 2
            if h >= 1:
                pl.semaphore_wait(readyA.at[r], 1)
            rdmaA = pltpu.make_async_remote_copy(
                src_ref=commA.at[s], dst_ref=commA.at[r],
                send_sem=sendA.at[s], recv_sem=recvA.at[r],
                device_id=(right,), device_id_type=pl.DeviceIdType.MESH,
            )
            rdmaA.start()
            if h >= 1:
                pl.semaphore_wait(readyB.at[r], 1)
            rdmaB = pltpu.make_async_remote_copy(
                src_ref=commB.at[s], dst_ref=commB.at[r],
                send_sem=sendB.at[s], recv_sem=recvB.at[r],
                device_id=(left,), device_id_type=pl.DeviceIdType.MESH,
            )
            rdmaB.start()
            pA[:, :] = partial_half(lax.rem(p + 2 * N_DEV - 2 - h, N_DEV), 0)
            pB[:, :] = partial_half(lax.rem(p + 2 + h, N_DEV), nh)
            rdmaA.wait_send()
            rdmaB.wait_send()
            if h <= N_DEV - 3:
                pl.semaphore_signal(
                    readyA.at[s], inc=1,
                    device_id=(left,), device_id_type=pl.DeviceIdType.MESH,
                )
                pl.semaphore_signal(
                    readyB.at[s], inc=1,
                    device_id=(right,), device_id_type=pl.DeviceIdType.MESH,
                )
            rdmaA.wait_recv()
            rdmaB.wait_recv()
            if h < N_DEV - 2:
                commA[r, :, :] = (
                    commA[r, :, :].astype(jnp.float32) + pA[:, :]
                ).astype(jnp.bfloat16)
                commB[r, :, :] = (
                    commB[r, :, :].astype(jnp.float32) + pB[:, :]
                ).astype(jnp.bfloat16)
            else:
                out_ref[:, 0:nh] = commA[r, :, :].astype(jnp.float32) + pA[:, :]
                out_ref[:, nh:n] = commB[r, :, :].astype(jnp.float32) + pB[:, :]

    return pl.pallas_call(
        body,
        out_shape=jax.ShapeDtypeStruct((m_per, n), jnp.float32),
        in_specs=[
            pl.BlockSpec(memory_space=pltpu.VMEM),
            pl.BlockSpec(memory_space=pltpu.VMEM),
        ],
        out_specs=pl.BlockSpec(memory_space=pltpu.VMEM),
        scratch_shapes=[
            pltpu.VMEM((2, m_per, nh), jnp.bfloat16),
            pltpu.VMEM((2, m_per, nh), jnp.bfloat16),
            pltpu.VMEM((m_per, nh), jnp.float32),
            pltpu.VMEM((m_per, nh), jnp.float32),
            pltpu.SemaphoreType.DMA((2,)),
            pltpu.SemaphoreType.DMA((2,)),
            pltpu.SemaphoreType.REGULAR((2,)),
            pltpu.SemaphoreType.DMA((2,)),
            pltpu.SemaphoreType.DMA((2,)),
            pltpu.SemaphoreType.REGULAR((2,)),
        ],
        compiler_params=pltpu.CompilerParams(
            collective_id=0,
            vmem_limit_bytes=64 * 1024 * 1024,
        ),
    )(A16, B16)


# device time: 237810 ns/iter; 1.6720x vs baseline; 1.0106x over previous
import jax
import jax.numpy as jnp
from jax import lax
from jax.experimental import pallas as pl
from jax.experimental.pallas import tpu as pltpu

N_DEV = 8
N_STREAMS = 4


def kernel(A, B):
    m, k_per = A.shape
    _, n = B.shape
    m_per = m // N_DEV
    nc = n // N_STREAMS

    A16 = A.astype(jnp.bfloat16)
    B16 = B.astype(jnp.bfloat16)

    def body(a_ref, b_ref, out_ref, comm, pbuf, send_sems, recv_sems,
             ready_sems):
        d = lax.axis_index("i")

        def ring_map(q):
            return jnp.where(q < 4, q, 11 - q)

        p = ring_map(d)
        right = ring_map(lax.rem(p + 1, N_DEV))
        left = ring_map(lax.rem(p + N_DEV - 1, N_DEV))

        streams = [(0 * nc, True), (2 * nc, False), (1 * nc, True),
                   (3 * nc, False)]

        barrier_sem = pltpu.get_barrier_semaphore()
        for nbr in (left, right):
            pl.semaphore_signal(
                barrier_sem, inc=1,
                device_id=(nbr,), device_id_type=pl.DeviceIdType.MESH,
            )
        pl.semaphore_wait(barrier_sem, 2)

        def partial(q, col0):
            row = ring_map(q) * m_per
            a = a_ref[pl.ds(row, m_per), :]
            return jnp.dot(a, b_ref[:, col0:col0 + nc],
                           preferred_element_type=jnp.float32)

        def chunk_pos(si_cw, h):
            if si_cw:
                return lax.rem(p + 2 * N_DEV - 2 - h, N_DEV)
            return lax.rem(p + 2 + h, N_DEV)

        for si, (col0, cw) in enumerate(streams):
            comm[si, 0, :, :] = partial(chunk_pos(cw, -1),
                                        col0).astype(jnp.bfloat16)

        for h in range(N_DEV - 1):
            s = h % 2
            r = (h + 1) % 2
            rdmas = []
            for si, (col0, cw) in enumerate(streams):
                if h >= 1:
                    pl.semaphore_wait(ready_sems.at[si, r], 1)
                rdma = pltpu.make_async_remote_copy(
                    src_ref=comm.at[si, s], dst_ref=comm.at[si, r],
                    send_sem=send_sems.at[si, s],
                    recv_sem=recv_sems.at[si, r],
                    device_id=(right if cw else left,),
                    device_id_type=pl.DeviceIdType.MESH,
                )
                rdma.start()
                rdmas.append(rdma)
            for si, (col0, cw) in enumerate(streams):
                pbuf[si, :, :] = partial(chunk_pos(cw, h), col0)
            for si, (col0, cw) in enumerate(streams):
                rdmas[si].wait_send()
                if h <= N_DEV - 3:
                    pl.semaphore_signal(
                        ready_sems.at[si, s], inc=1,
                        device_id=(left if cw else right,),
                        device_id_type=pl.DeviceIdType.MESH,
                    )
            for si, (col0, cw) in enumerate(streams):
                rdmas[si].wait_recv()
                if h < N_DEV - 2:
                    comm[si, r, :, :] = (
                        comm[si, r, :, :].astype(jnp.float32)
                        + pbuf[si, :, :]
                    ).astype(jnp.bfloat16)
                else:
                    out_ref[:, col0:col0 + nc] = (
                        comm[si, r, :, :].astype(jnp.float32)
                        + pbuf[si, :, :]
                    )

    return pl.pallas_call(
        body,
        out_shape=jax.ShapeDtypeStruct((m_per, n), jnp.float32),
        in_specs=[
            pl.BlockSpec(memory_space=pltpu.VMEM),
            pl.BlockSpec(memory_space=pltpu.VMEM),
        ],
        out_specs=pl.BlockSpec(memory_space=pltpu.VMEM),
        scratch_shapes=[
            pltpu.VMEM((N_STREAMS, 2, m_per, nc), jnp.bfloat16),
            pltpu.VMEM((N_STREAMS, m_per, nc), jnp.float32),
            pltpu.SemaphoreType.DMA((N_STREAMS, 2)),
            pltpu.SemaphoreType.DMA((N_STREAMS, 2)),
            pltpu.SemaphoreType.REGULAR((N_STREAMS, 2)),
        ],
        compiler_params=pltpu.CompilerParams(
            collective_id=0,
            vmem_limit_bytes=64 * 1024 * 1024,
        ),
    )(A16, B16)


# device time: 202154 ns/iter; 1.9669x vs baseline; 1.1764x over previous
import jax
import jax.numpy as jnp
from jax import lax
from jax.experimental import pallas as pl
from jax.experimental.pallas import tpu as pltpu

N_DEV = 8
N_STREAMS = 4


def kernel(A, B):
    m, k_per = A.shape
    _, n = B.shape
    m_per = m // N_DEV
    nc = n // N_STREAMS

    def body(a_hbm, b_hbm, out_ref, b16, bstage, astage, comm, pbuf,
             dma_sems, send_sems, recv_sems, ready_sems):
        d = lax.axis_index("i")

        def ring_map(q):
            return jnp.where(q < 4, q, 11 - q)

        p = ring_map(d)
        right = ring_map(lax.rem(p + 1, N_DEV))
        left = ring_map(lax.rem(p + N_DEV - 1, N_DEV))

        streams = [(0 * nc, True), (2 * nc, False), (1 * nc, True),
                   (3 * nc, False)]

        def chunk_pos(cw, h):
            if cw:
                return lax.rem(p + 2 * N_DEV - 2 - h, N_DEV)
            return lax.rem(p + 2 + h, N_DEV)

        def a_chunk_dma(q, dir_idx):
            row = ring_map(q) * m_per
            return pltpu.make_async_copy(
                a_hbm.at[pl.ds(row, m_per), :],
                astage.at[dir_idx],
                dma_sems.at[dir_idx],
            )

        def dot16(dir_idx, col0):
            return jnp.dot(astage[dir_idx, :, :].astype(jnp.bfloat16),
                           b16[:, col0:col0 + nc],
                           preferred_element_type=jnp.float32)

        def hop_rdma(si, cw, s, r):
            return pltpu.make_async_remote_copy(
                src_ref=comm.at[si, s], dst_ref=comm.at[si, r],
                send_sem=send_sems.at[si, s],
                recv_sem=recv_sems.at[si, r],
                device_id=(right if cw else left,),
                device_id_type=pl.DeviceIdType.MESH,
            )

        barrier_sem = pltpu.get_barrier_semaphore()
        for nbr in (left, right):
            pl.semaphore_signal(
                barrier_sem, inc=1,
                device_id=(nbr,), device_id_type=pl.DeviceIdType.MESH,
            )
        pl.semaphore_wait(barrier_sem, 2)

        a_chunk_dma(chunk_pos(True, -1), 0).start()
        a_chunk_dma(chunk_pos(False, -1), 1).start()
        waited = [False, False]
        for si, (col0, cw) in enumerate(streams):
            bdma = pltpu.make_async_copy(
                b_hbm.at[:, pl.ds(col0, nc)], bstage, dma_sems.at[2])
            bdma.start()
            bdma.wait()
            b16[:, col0:col0 + nc] = bstage[:, :].astype(jnp.bfloat16)
            dir_idx = 0 if cw else 1
            if not waited[dir_idx]:
                a_chunk_dma(chunk_pos(cw, -1), dir_idx).wait()
                waited[dir_idx] = True
            comm[si, 0, :, :] = dot16(dir_idx, col0).astype(jnp.bfloat16)
            hop_rdma(si, cw, 0, 1).start()
        a_chunk_dma(chunk_pos(True, 0), 0).start()
        a_chunk_dma(chunk_pos(False, 0), 1).start()

        for h in range(N_DEV - 1):
            s = h % 2
            r = (h + 1) % 2
            rdmas = []
            for si, (col0, cw) in enumerate(streams):
                rdma = hop_rdma(si, cw, s, r)
                if h >= 1:
                    pl.semaphore_wait(ready_sems.at[si, r], 1)
                    rdma.start()
                rdmas.append(rdma)
            pltpu.make_async_copy(
                a_hbm.at[pl.ds(0, m_per), :], astage.at[0],
                dma_sems.at[0]).wait()
            pltpu.make_async_copy(
                a_hbm.at[pl.ds(0, m_per), :], astage.at[1],
                dma_sems.at[1]).wait()
            for si, (col0, cw) in enumerate(streams):
                pbuf[si, :, :] = dot16(0 if cw else 1, col0)
            if h < N_DEV - 2:
                a_chunk_dma(chunk_pos(True, h + 1), 0).start()
                a_chunk_dma(chunk_pos(False, h + 1), 1).start()
            for si, (col0, cw) in enumerate(streams):
                rdmas[si].wait_send()
                if h <= N_DEV - 3:
                    pl.semaphore_signal(
                        ready_sems.at[si, s], inc=1,
                        device_id=(left if cw else right,),
                        device_id_type=pl.DeviceIdType.MESH,
                    )
            for si, (col0, cw) in enumerate(streams):
                rdmas[si].wait_recv()
                if h < N_DEV - 2:
                    comm[si, r, :, :] = (
                        comm[si, r, :, :].astype(jnp.float32)
                        + pbuf[si, :, :]
                    ).astype(jnp.bfloat16)
                else:
                    out_ref[:, col0:col0 + nc] = (
                        comm[si, r, :, :].astype(jnp.float32)
                        + pbuf[si, :, :]
                    )

    return pl.pallas_call(
        body,
        out_shape=jax.ShapeDtypeStruct((m_per, n), jnp.float32),
        in_specs=[
            pl.BlockSpec(memory_space=pl.ANY),
            pl.BlockSpec(memory_space=pl.ANY),
        ],
        out_specs=pl.BlockSpec(memory_space=pltpu.VMEM),
        scratch_shapes=[
            pltpu.VMEM((k_per, n), jnp.bfloat16),
            pltpu.VMEM((k_per, nc), jnp.float32),
            pltpu.VMEM((2, m_per, k_per), jnp.float32),
            pltpu.VMEM((N_STREAMS, 2, m_per, nc), jnp.bfloat16),
            pltpu.VMEM((N_STREAMS, m_per, nc), jnp.float32),
            pltpu.SemaphoreType.DMA((3,)),
            pltpu.SemaphoreType.DMA((N_STREAMS, 2)),
            pltpu.SemaphoreType.DMA((N_STREAMS, 2)),
            pltpu.SemaphoreType.REGULAR((N_STREAMS, 2)),
        ],
        compiler_params=pltpu.CompilerParams(
            collective_id=0,
            vmem_limit_bytes=64 * 1024 * 1024,
        ),
    )(A, B)


# device time: 196067 ns/iter; 2.0279x vs baseline; 1.0310x over previous
import jax
import jax.numpy as jnp
from jax import lax
from jax.experimental import pallas as pl
from jax.experimental.pallas import tpu as pltpu

N_DEV = 8
N_STREAMS = 4


def kernel(A, B):
    m, k_per = A.shape
    _, n = B.shape
    m_per = m // N_DEV
    nc = n // N_STREAMS

    def body(a_hbm, b_hbm, out_ref, b16, bstage, astage, comm, pbuf,
             dma_sems, send_sems, recv_sems, ready_sems):
        d = lax.axis_index("i")

        def ring_map(q):
            return jnp.where(q < 4, q, 11 - q)

        p = ring_map(d)
        right = ring_map(lax.rem(p + 1, N_DEV))
        left = ring_map(lax.rem(p + N_DEV - 1, N_DEV))

        streams = [(0 * nc, True), (2 * nc, False), (1 * nc, True),
                   (3 * nc, False)]

        def chunk_pos(cw, h):
            if cw:
                return lax.rem(p + 2 * N_DEV - 2 - h, N_DEV)
            return lax.rem(p + 2 + h, N_DEV)

        def a_chunk_dma(q, dir_idx):
            row = ring_map(q) * m_per
            return pltpu.make_async_copy(
                a_hbm.at[pl.ds(row, m_per), :],
                astage.at[dir_idx],
                dma_sems.at[dir_idx],
            )

        def dot16(dir_idx, col0):
            return jnp.dot(astage[dir_idx, :, :].astype(jnp.bfloat16),
                           b16[:, col0:col0 + nc],
                           preferred_element_type=jnp.float32)

        def hop_rdma(si, cw, s, r):
            return pltpu.make_async_remote_copy(
                src_ref=comm.at[si, s], dst_ref=comm.at[si, r],
                send_sem=send_sems.at[si, s],
                recv_sem=recv_sems.at[si, r],
                device_id=(right if cw else left,),
                device_id_type=pl.DeviceIdType.MESH,
            )

        barrier_sem = pltpu.get_barrier_semaphore()
        for nbr in (left, right):
            pl.semaphore_signal(
                barrier_sem, inc=1,
                device_id=(nbr,), device_id_type=pl.DeviceIdType.MESH,
            )
        pl.semaphore_wait(barrier_sem, 2)

        a_chunk_dma(chunk_pos(True, -1), 0).start()
        a_chunk_dma(chunk_pos(False, -1), 1).start()
        waited = [False, False]
        for si, (col0, cw) in enumerate(streams):
            bdma = pltpu.make_async_copy(
                b_hbm.at[:, pl.ds(col0, nc)], bstage, dma_sems.at[2])
            bdma.start()
            bdma.wait()
            b16[:, col0:col0 + nc] = bstage[:, :].astype(jnp.bfloat16)
            dir_idx = 0 if cw else 1
            if not waited[dir_idx]:
                a_chunk_dma(chunk_pos(cw, -1), dir_idx).wait()
                waited[dir_idx] = True
            comm[si, 0, :, :] = dot16(dir_idx, col0).astype(jnp.bfloat16)
            hop_rdma(si, cw, 0, 1).start()
        a_chunk_dma(chunk_pos(True, 0), 0).start()
        a_chunk_dma(chunk_pos(False, 0), 1).start()

        for h in range(N_DEV - 1):
            s = h % 2
            r = (h + 1) % 2
            pltpu.make_async_copy(
                a_hbm.at[pl.ds(0, m_per), :], astage.at[0],
                dma_sems.at[0]).wait()
            pltpu.make_async_copy(
                a_hbm.at[pl.ds(0, m_per), :], astage.at[1],
                dma_sems.at[1]).wait()
            for si, (col0, cw) in enumerate(streams):
                pbuf[si, :, :] = dot16(0 if cw else 1, col0)
            if h < N_DEV - 2:
                a_chunk_dma(chunk_pos(True, h + 1), 0).start()
                a_chunk_dma(chunk_pos(False, h + 1), 1).start()
            for si, (col0, cw) in enumerate(streams):
                hop_rdma(si, cw, s, r).wait_send()
                if h <= N_DEV - 3:
                    pl.semaphore_signal(
                        ready_sems.at[si, s], inc=1,
                        device_id=(left if cw else right,),
                        device_id_type=pl.DeviceIdType.MESH,
                    )
            for si, (col0, cw) in enumerate(streams):
                hop_rdma(si, cw, s, r).wait_recv()
                if h < N_DEV - 2:
                    comm[si, r, :, :] = (
                        comm[si, r, :, :].astype(jnp.float32)
                        + pbuf[si, :, :]
                    ).astype(jnp.bfloat16)
                    pl.semaphore_wait(ready_sems.at[si, s], 1)
                    hop_rdma(si, cw, r, s).start()
                else:
                    out_ref[:, col0:col0 + nc] = (
                        comm[si, r, :, :].astype(jnp.float32)
                        + pbuf[si, :, :]
                    )

    return pl.pallas_call(
        body,
        out_shape=jax.ShapeDtypeStruct((m_per, n), jnp.float32),
        in_specs=[
            pl.BlockSpec(memory_space=pl.ANY),
            pl.BlockSpec(memory_space=pl.ANY),
        ],
        out_specs=pl.BlockSpec(memory_space=pltpu.VMEM),
        scratch_shapes=[
            pltpu.VMEM((k_per, n), jnp.bfloat16),
            pltpu.VMEM((k_per, nc), jnp.float32),
            pltpu.VMEM((2, m_per, k_per), jnp.float32),
            pltpu.VMEM((N_STREAMS, 2, m_per, nc), jnp.bfloat16),
            pltpu.VMEM((N_STREAMS, m_per, nc), jnp.float32),
            pltpu.SemaphoreType.DMA((3,)),
            pltpu.SemaphoreType.DMA((N_STREAMS, 2)),
            pltpu.SemaphoreType.DMA((N_STREAMS, 2)),
            pltpu.SemaphoreType.REGULAR((N_STREAMS, 2)),
        ],
        compiler_params=pltpu.CompilerParams(
            collective_id=0,
            vmem_limit_bytes=64 * 1024 * 1024,
        ),
    )(A, B)


# device time: 170691 ns/iter; 2.3294x vs baseline; 1.1487x over previous
import jax
import jax.numpy as jnp
from jax import lax
from jax.experimental import pallas as pl
from jax.experimental.pallas import tpu as pltpu

N_DEV = 8
N_SYS = 3
N_STEPS = 7

E = [
    [1, 2, 1, 4, 1, 2, 1],
    [2, 4, 2, 1, 2, 4, 2],
    [4, 1, 4, 2, 4, 1, 4],
]
MX = [
    [4, 5, 7, 6, 2, 3, 1, 0],
    [1, 3, 7, 5, 4, 6, 2, 0],
    [2, 6, 7, 3, 1, 5, 4, 0],
]
COLS = [(0, 1408), (1408, 1408), (2816, 1280)]
NCMAX = 1408


def kernel(A, B):
    m, k_per = A.shape
    _, n = B.shape
    m_per = m // N_DEV

    def body(a_hbm, b_hbm, out_ref, b16, bstage, astage, comm, pbuf,
             dma_sems, send_sems, recv_sems, ready_sems):
        d = lax.axis_index("i")

        def to_vertex(q):
            return q ^ ((q >> 1) & 1)

        v = to_vertex(d)

        def nbr(sys, t):
            return to_vertex(v ^ E[sys][t])

        def chunk_row(sys, t):
            return to_vertex(v ^ MX[sys][t]) * m_per

        def a_chunk_dma(sys, t):
            return pltpu.make_async_copy(
                a_hbm.at[pl.ds(chunk_row(sys, t), m_per), :],
                astage.at[sys],
                dma_sems.at[sys],
            )

        def dot16(sys):
            col0, w = COLS[sys]
            return jnp.dot(astage[sys, :, :].astype(jnp.bfloat16),
                           b16[:, col0:col0 + w],
                           preferred_element_type=jnp.float32)

        def step_rdma(sys, t, s, r):
            return pltpu.make_async_remote_copy(
                src_ref=comm.at[sys, s], dst_ref=comm.at[sys, r],
                send_sem=send_sems.at[sys, s],
                recv_sem=recv_sems.at[sys, r],
                device_id=(nbr(sys, t),),
                device_id_type=pl.DeviceIdType.MESH,
            )

        barrier_sem = pltpu.get_barrier_semaphore()
        for e in (1, 2, 4):
            pl.semaphore_signal(
                barrier_sem, inc=1,
                device_id=(to_vertex(v ^ e),),
                device_id_type=pl.DeviceIdType.MESH,
            )
        pl.semaphore_wait(barrier_sem, 3)

        for sys in range(N_SYS):
            a_chunk_dma(sys, 0).start()
        for sys in range(N_SYS):
            col0, w = COLS[sys]
            for off in range(0, w, 512):
                pw = min(512, w - off)
                bdma = pltpu.make_async_copy(
                    b_hbm.at[:, pl.ds(col0 + off, pw)],
                    bstage.at[:, pl.ds(0, pw)],
                    dma_sems.at[N_SYS])
                bdma.start()
                bdma.wait()
                b16[:, col0 + off:col0 + off + pw] = (
                    bstage[:, 0:pw].astype(jnp.bfloat16))
            a_chunk_dma(sys, 0).wait()
            comm[sys, 0, :, 0:w] = dot16(sys).astype(jnp.bfloat16)
            step_rdma(sys, 0, 0, 1).start()
        for sys in range(N_SYS):
            a_chunk_dma(sys, 1).start()

        for t in range(N_STEPS):
            s = t % 2
            r = (t + 1) % 2
            for sys in range(N_SYS):
                pltpu.make_async_copy(
                    a_hbm.at[pl.ds(0, m_per), :], astage.at[sys],
                    dma_sems.at[sys]).wait()
            for sys in range(N_SYS):
                col0, w = COLS[sys]
                pbuf[sys, :, 0:w] = dot16(sys)
            if t < N_STEPS - 1:
                for sys in range(N_SYS):
                    a_chunk_dma(sys, t + 2).start()
            for sys in range(N_SYS):
                step_rdma(sys, t, s, r).wait_send()
                if t <= N_STEPS - 2:
                    pl.semaphore_signal(
                        ready_sems.at[sys, s], inc=1,
                        device_id=(nbr(sys, t + 1),),
                        device_id_type=pl.DeviceIdType.MESH,
                    )
            for sys in range(N_SYS):
                col0, w = COLS[sys]
                step_rdma(sys, t, s, r).wait_recv()
                if t < N_STEPS - 1:
                    comm[sys, r, :, 0:w] = (
                        comm[sys, r, :, 0:w].astype(jnp.float32)
                        + pbuf[sys, :, 0:w]
                    ).astype(jnp.bfloat16)
                    pl.semaphore_wait(ready_sems.at[sys, s], 1)
                    step_rdma(sys, t + 1, r, s).start()
                else:
                    out_ref[:, col0:col0 + w] = (
                        comm[sys, r, :, 0:w].astype(jnp.float32)
                        + pbuf[sys, :, 0:w]
                    )

    return pl.pallas_call(
        body,
        out_shape=jax.ShapeDtypeStruct((m_per, n), jnp.float32),
        in_specs=[
            pl.BlockSpec(memory_space=pl.ANY),
            pl.BlockSpec(memory_space=pl.ANY),
        ],
        out_specs=pl.BlockSpec(memory_space=pltpu.VMEM),
        scratch_shapes=[
            pltpu.VMEM((k_per, n), jnp.bfloat16),
            pltpu.VMEM((k_per, 512), jnp.float32),
            pltpu.VMEM((N_SYS, m_per, k_per), jnp.float32),
            pltpu.VMEM((N_SYS, 2, m_per, NCMAX), jnp.bfloat16),
            pltpu.VMEM((N_SYS, m_per, NCMAX), jnp.float32),
            pltpu.SemaphoreType.DMA((N_SYS + 1,)),
            pltpu.SemaphoreType.DMA((N_SYS, 2)),
            pltpu.SemaphoreType.DMA((N_SYS, 2)),
            pltpu.SemaphoreType.REGULAR((N_SYS, 2)),
        ],
        compiler_params=pltpu.CompilerParams(
            collective_id=0,
            vmem_limit_bytes=64 * 1024 * 1024,
        ),
    )(A, B)
